# Initial kernel scaffold; baseline (speedup 1.0000x reference)
import jax
import jax.numpy as jnp
from jax.experimental import pallas as pl


def kernel(x, edge_index, c1_W1, c1_b1, c1_W2, c1_b2, c1_W3, c1_b3, bn1_g, bn1_b, c2_W1, c2_b1, c2_W2, c2_b2, c2_W3, c2_b3, bn2_g, bn2_b, c3_W1, c3_b1, c3_W2, c3_b2, c3_W3, c3_b3, bn3_g, bn3_b, lin_W, lin_b, cls_W, cls_b):
    raise NotImplementedError("write your pallas kernel here")


# speedup vs baseline: 2.7815x; 2.7815x over previous
import functools

import jax
import jax.numpy as jnp
from jax import lax
from jax.experimental import pallas as pl
from jax.experimental.pallas import tpu as pltpu
from jax.experimental.pallas import tpu_sc as plsc

_N = 10000
_E = 320000
_D = 128
_OUT = 64

_CHUNK = 128
_ROWS = 2560
_EP = _ROWS * _CHUNK
_NP = 10240
_ROWS_PER_TILE = _ROWS // 32
_ACC_PER_TILE = _NP // 16

_BLK = 2000
_NBLK = _N // _BLK


def _segsum_sc(x, src2d, dst2d):
    mesh = plsc.VectorSubcoreMesh(core_axis_name="c", subcore_axis_name="s")

    @functools.partial(
        pl.kernel,
        out_type=jax.ShapeDtypeStruct((2, _NP, _D), jnp.float32),
        mesh=mesh,
        scratch_types=[
            pltpu.VMEM((_ROWS_PER_TILE, _CHUNK), jnp.int32),
            pltpu.VMEM((_ROWS_PER_TILE, _CHUNK), jnp.int32),
            pltpu.VMEM((_CHUNK, _D), jnp.float32),
            pltpu.VMEM_SHARED((_NP, _D), jnp.float32),
            pltpu.SemaphoreType.DMA,
        ],
    )
    def k(x_hbm, src_hbm, dst_hbm, out_hbm, sidx, didx, rows, acc, sem):
        c = lax.axis_index("c")
        s = lax.axis_index("s")

        def _zrow(i, carry):
            for j in range(_D // 16):
                rows[i, pl.ds(j * 16, 16)] = jnp.zeros((16,), jnp.float32)
            return carry
        lax.fori_loop(0, _CHUNK, _zrow, 0)

        def _zacc(t, carry):
            pltpu.sync_copy(rows, acc.at[pl.ds(s * _ACC_PER_TILE + t * _CHUNK,
                                               _CHUNK)])
            return carry
        lax.fori_loop(0, _ACC_PER_TILE // _CHUNK, _zacc, 0)
        plsc.subcore_barrier()

        base = c * (_ROWS // 2) + s * _ROWS_PER_TILE
        pltpu.sync_copy(src_hbm.at[pl.ds(base, _ROWS_PER_TILE)], sidx)
        pltpu.sync_copy(dst_hbm.at[pl.ds(base, _ROWS_PER_TILE)], didx)

        def _body(i, carry):
            pltpu.async_copy(x_hbm.at[sidx.at[i]], rows, sem).wait()
            pltpu.sync_copy(rows, acc.at[didx.at[i]], add=True)
            return carry
        lax.fori_loop(0, _ROWS_PER_TILE, _body, 0)
        plsc.subcore_barrier()

        def _out(t, carry):
            off = s * _ACC_PER_TILE + t * _CHUNK
            pltpu.sync_copy(acc.at[pl.ds(off, _CHUNK)], rows)
            pltpu.sync_copy(rows, out_hbm.at[c, pl.ds(off, _CHUNK)])
            return carry
        lax.fori_loop(0, _ACC_PER_TILE // _CHUNK, _out, 0)

    return k(x, src2d, dst2d)


def _mlp_body(x_ref, pa_ref, pb_ref, w1, b1, w2, b2, w3, b3,
              h_ref, sum_ref):
    i = pl.program_id(0)
    h0 = x_ref[...] + pa_ref[0] + pb_ref[0]
    h = jnp.maximum(jnp.dot(h0, w1[...], preferred_element_type=jnp.float32)
                    + b1[...], 0.0)
    h = jnp.maximum(jnp.dot(h, w2[...], preferred_element_type=jnp.float32)
                    + b2[...], 0.0)
    h = jnp.dot(h, w3[...], preferred_element_type=jnp.float32) + b3[...]
    h_ref[...] = h

    @pl.when(i == 0)
    def _():
        sum_ref[...] = jnp.zeros_like(sum_ref)

    sum_ref[...] += jnp.broadcast_to(jnp.sum(h, axis=0, keepdims=True),
                                     sum_ref.shape)


def _mlp_stats(x, part, w1, b1, w2, b2, w3, b3):
    return pl.pallas_call(
        _mlp_body,
        grid=(_NBLK,),
        in_specs=[
            pl.BlockSpec((_BLK, _D), lambda i: (i, 0)),
            pl.BlockSpec((1, _BLK, _D), lambda i: (0, i, 0)),
            pl.BlockSpec((1, _BLK, _D), lambda i: (1, i, 0)),
            pl.BlockSpec((_D, _D), lambda i: (0, 0)),
            pl.BlockSpec((1, _D), lambda i: (0, 0)),
            pl.BlockSpec((_D, _D), lambda i: (0, 0)),
            pl.BlockSpec((1, _D), lambda i: (0, 0)),
            pl.BlockSpec((_D, _D), lambda i: (0, 0)),
            pl.BlockSpec((1, _D), lambda i: (0, 0)),
        ],
        out_specs=[
            pl.BlockSpec((_BLK, _D), lambda i: (i, 0)),
            pl.BlockSpec((8, _D), lambda i: (0, 0)),
        ],
        out_shape=[
            jax.ShapeDtypeStruct((_N, _D), jnp.float32),
            jax.ShapeDtypeStruct((8, _D), jnp.float32),
        ],
    )(x, part, part, w1, b1, w2, b2, w3, b3)


def _sq_body(h_ref, sum_ref, sq_ref):
    i = pl.program_id(0)
    mu = sum_ref[0:1, :] * (1.0 / _N)
    c = h_ref[...] - mu

    @pl.when(i == 0)
    def _():
        sq_ref[...] = jnp.zeros_like(sq_ref)

    sq_ref[...] += jnp.broadcast_to(jnp.sum(c * c, axis=0, keepdims=True),
                                    sq_ref.shape)


def _sq_stats(h, ssum):
    return pl.pallas_call(
        _sq_body,
        grid=(_NBLK,),
        in_specs=[
            pl.BlockSpec((_BLK, _D), lambda i: (i, 0)),
            pl.BlockSpec((8, _D), lambda i: (0, 0)),
        ],
        out_specs=pl.BlockSpec((8, _D), lambda i: (0, 0)),
        out_shape=jax.ShapeDtypeStruct((8, _D), jnp.float32),
    )(h, ssum)


def _bn_body(relu, h_ref, sum_ref, sq_ref, g_ref, b_ref, o_ref):
    mu = sum_ref[0:1, :] * (1.0 / _N)
    var = sq_ref[0:1, :] * (1.0 / _N)
    y = (h_ref[...] - mu) * lax.rsqrt(var + 1e-5) * g_ref[...] + b_ref[...]
    if relu:
        y = jnp.maximum(y, 0.0)
    o_ref[...] = y


def _bn_apply(h, ssum, ssq, g, b, relu):
    return pl.pallas_call(
        functools.partial(_bn_body, relu),
        grid=(_NBLK,),
        in_specs=[
            pl.BlockSpec((_BLK, _D), lambda i: (i, 0)),
            pl.BlockSpec((8, _D), lambda i: (0, 0)),
            pl.BlockSpec((8, _D), lambda i: (0, 0)),
            pl.BlockSpec((1, _D), lambda i: (0, 0)),
            pl.BlockSpec((1, _D), lambda i: (0, 0)),
        ],
        out_specs=pl.BlockSpec((_BLK, _D), lambda i: (i, 0)),
        out_shape=jax.ShapeDtypeStruct((_N, _D), jnp.float32),
    )(h, ssum, ssq, g, b)


def _final_body(h1_ref, h2_ref, h3_ref, wa, wb, wc, lb, cw, cb,
                h_ref, logit_ref):
    h = (jnp.dot(h1_ref[...], wa[0], preferred_element_type=jnp.float32)
         + jnp.dot(h2_ref[...], wb[0], preferred_element_type=jnp.float32)
         + jnp.dot(h3_ref[...], wc[0], preferred_element_type=jnp.float32)
         + lb[...])
    h_ref[...] = h
    logit_ref[...] = jnp.dot(h, cw[...], preferred_element_type=jnp.float32) \
        + cb[...]


def _final(h1, h2, h3, lin_W, lin_b, cls_W, cls_b):
    lw3 = lin_W.reshape(3, _D, _D)
    return pl.pallas_call(
        _final_body,
        grid=(_NBLK,),
        in_specs=[
            pl.BlockSpec((_BLK, _D), lambda i: (i, 0)),
            pl.BlockSpec((_BLK, _D), lambda i: (i, 0)),
            pl.BlockSpec((_BLK, _D), lambda i: (i, 0)),
            pl.BlockSpec((1, _D, _D), lambda i: (0, 0, 0)),
            pl.BlockSpec((1, _D, _D), lambda i: (1, 0, 0)),
            pl.BlockSpec((1, _D, _D), lambda i: (2, 0, 0)),
            pl.BlockSpec((1, _D), lambda i: (0, 0)),
            pl.BlockSpec((_D, _OUT), lambda i: (0, 0)),
            pl.BlockSpec((1, _OUT), lambda i: (0, 0)),
        ],
        out_specs=[
            pl.BlockSpec((_BLK, _D), lambda i: (i, 0)),
            pl.BlockSpec((_BLK, _OUT), lambda i: (i, 0)),
        ],
        out_shape=[
            jax.ShapeDtypeStruct((_N, _D), jnp.float32),
            jax.ShapeDtypeStruct((_N, _OUT), jnp.float32),
        ],
    )(h1, h2, h3, lw3, lw3, lw3, lin_b, cls_W, cls_b)


def kernel(x, edge_index,
           c1_W1, c1_b1, c1_W2, c1_b2, c1_W3, c1_b3, bn1_g, bn1_b,
           c2_W1, c2_b1, c2_W2, c2_b2, c2_W3, c2_b3, bn2_g, bn2_b,
           c3_W1, c3_b1, c3_W2, c3_b2, c3_W3, c3_b3, bn3_g, bn3_b,
           lin_W, lin_b, cls_W, cls_b):
    src = edge_index[0]
    dst = edge_index[1]
    pad = _EP - _E
    src2d = jnp.concatenate([src, jnp.zeros((pad,), jnp.int32)]) \
        .reshape(_ROWS, _CHUNK)
    dst2d = jnp.concatenate([dst, jnp.full((pad,), _N, jnp.int32)]) \
        .reshape(_ROWS, _CHUNK)

    r2 = lambda v: v.reshape(1, -1)
    layers = [
        (c1_W1, r2(c1_b1), c1_W2, r2(c1_b2), c1_W3, r2(c1_b3),
         r2(bn1_g), r2(bn1_b), True),
        (c2_W1, r2(c2_b1), c2_W2, r2(c2_b2), c2_W3, r2(c2_b3),
         r2(bn2_g), r2(bn2_b), True),
        (c3_W1, r2(c3_b1), c3_W2, r2(c3_b2), c3_W3, r2(c3_b3),
         r2(bn3_g), r2(bn3_b), False),
    ]

    hs = []
    h = x
    for (w1, b1, w2, b2, w3, b3, g, b, relu) in layers:
        part = _segsum_sc(h, src2d, dst2d)
        h_pre, ssum = _mlp_stats(h, part, w1, b1, w2, b2, w3, b3)
        ssq = _sq_stats(h_pre, ssum)
        h = _bn_apply(h_pre, ssum, ssq, g, b, relu)
        hs.append(h)

    h_out, logits = _final(hs[0], hs[1], hs[2], lin_W, r2(lin_b),
                           cls_W, r2(cls_b))
    return (logits, h_out)
